# Initial kernel scaffold; baseline (speedup 1.0000x reference)
import jax
import jax.numpy as jnp
from jax.experimental import pallas as pl


def kernel(x, edge_index, Wg1, bg1, We1, be1, bias1, Wg2, bg2, We2, be2, bias2, W3, bias3):
    raise NotImplementedError("write your pallas kernel here")


# speedup vs baseline: 5.4546x; 5.4546x over previous
import functools

import jax
import jax.numpy as jnp
from jax import lax
from jax.experimental import pallas as pl
from jax.experimental.pallas import tpu as pltpu
from jax.experimental.pallas import tpu_sc as plsc

N = 10000
E = 320000
D = 128
NEXP = 16

NC = 2
NS = 16
NW = NC * NS
EPW = E // NW
EDGE_B = 128
NB_FULL = EPW // EDGE_B
TAIL = EPW - NB_FULL * EDGE_B
ROWS_MAIN = 632
ROWS_LAST = N - ROWS_MAIN * (NS - 1)

RB = 1000


def _gate_and_expert(X, Wg, bg, We, be):
    gate = jnp.dot(X, Wg, preferred_element_type=jnp.float32) + bg
    idx = jnp.argmax(gate, axis=1)[:, None]
    acc = jnp.zeros((RB, D), jnp.float32)
    for e in range(NEXP):
        pe = jnp.dot(X, We[e], preferred_element_type=jnp.float32) + be[e][None, :]
        acc = acc + jnp.where(idx == e, 1.0, 0.0) * pe
    m = jnp.mean(gate, axis=1, keepdims=True)
    d = gate - m
    var = jnp.sum(d * d, axis=1, keepdims=True) / (NEXP - 1)
    ssum = jnp.sum(jnp.sqrt(var))
    return acc, ssum


def _accum_scalar(ssum_ref, val):
    i = pl.program_id(0)
    v = jnp.full((1, 1), val, jnp.float32)

    @pl.when(i == 0)
    def _():
        ssum_ref[...] = v

    @pl.when(i > 0)
    def _():
        ssum_ref[...] = ssum_ref[...] + v


def _moe_first_body(x_ref, wg_ref, bg_ref, we_ref, be_ref, h_ref, ssum_ref):
    h, ssum = _gate_and_expert(x_ref[...], wg_ref[...], bg_ref[...],
                               we_ref, be_ref[...])
    h_ref[...] = h
    _accum_scalar(ssum_ref, ssum)


def _moe_next_body(p0_ref, p1_ref, bprev_ref, wg_ref, bg_ref, we_ref, be_ref,
                   h_ref, ssum_ref):
    X = jax.nn.relu(p0_ref[...] + p1_ref[...] + bprev_ref[...])
    h, ssum = _gate_and_expert(X, wg_ref[...], bg_ref[...], we_ref, be_ref[...])
    h_ref[...] = h
    _accum_scalar(ssum_ref, ssum)


def _final_mm_body(p0_ref, p1_ref, bprev_ref, w3_ref, h_ref):
    X = jax.nn.relu(p0_ref[...] + p1_ref[...] + bprev_ref[...])
    h_ref[...] = jnp.dot(X, w3_ref[...], preferred_element_type=jnp.float32)


def _combine_body(p0_ref, p1_ref, b_ref, out_ref):
    out_ref[...] = p0_ref[...] + p1_ref[...] + b_ref[...]


_row_spec = pl.BlockSpec((RB, D), lambda i: (i, 0))
_full2 = lambda shape: pl.BlockSpec(shape, lambda i: (0,) * len(shape))
_scalar_spec = pl.BlockSpec((1, 1), lambda i: (0, 0))


def _moe_first(x, Wg, bg, We, be):
    return pl.pallas_call(
        _moe_first_body,
        grid=(N // RB,),
        in_specs=[_row_spec, _full2((D, NEXP)), _full2((1, NEXP)),
                  _full2((NEXP, D, D)), _full2((NEXP, D))],
        out_specs=[_row_spec, _scalar_spec],
        out_shape=[jax.ShapeDtypeStruct((N, D), jnp.float32),
                   jax.ShapeDtypeStruct((1, 1), jnp.float32)],
    )(x, Wg, bg.reshape(1, NEXP), We, be)


def _moe_next(p0, p1, bprev, Wg, bg, We, be):
    return pl.pallas_call(
        _moe_next_body,
        grid=(N // RB,),
        in_specs=[_row_spec, _row_spec, _full2((1, D)), _full2((D, NEXP)),
                  _full2((1, NEXP)), _full2((NEXP, D, D)), _full2((NEXP, D))],
        out_specs=[_row_spec, _scalar_spec],
        out_shape=[jax.ShapeDtypeStruct((N, D), jnp.float32),
                   jax.ShapeDtypeStruct((1, 1), jnp.float32)],
    )(p0, p1, bprev.reshape(1, D), Wg, bg.reshape(1, NEXP), We, be)


def _final_mm(p0, p1, bprev, W3):
    return pl.pallas_call(
        _final_mm_body,
        grid=(N // RB,),
        in_specs=[_row_spec, _row_spec, _full2((1, D)), _full2((D, D))],
        out_specs=_row_spec,
        out_shape=jax.ShapeDtypeStruct((N, D), jnp.float32),
    )(p0, p1, bprev.reshape(1, D), W3)


def _combine(p0, p1, b):
    return pl.pallas_call(
        _combine_body,
        grid=(N // RB,),
        in_specs=[_row_spec, _row_spec, _full2((1, D))],
        out_specs=_row_spec,
        out_shape=jax.ShapeDtypeStruct((N, D), jnp.float32),
    )(p0, p1, b.reshape(1, D))


_sc_mesh = plsc.VectorSubcoreMesh(core_axis_name="c", subcore_axis_name="s")


@functools.partial(
    pl.kernel,
    mesh=_sc_mesh,
    out_type=jax.ShapeDtypeStruct((NC * N, D), jnp.float32),
    scratch_types=[
        pltpu.VMEM((EDGE_B,), jnp.int32),
        pltpu.VMEM((EDGE_B,), jnp.int32),
        pltpu.VMEM((EDGE_B, D), jnp.float32),
        pltpu.VMEM((TAIL,), jnp.int32),
        pltpu.VMEM((TAIL,), jnp.int32),
        pltpu.VMEM((TAIL, D), jnp.float32),
        pltpu.VMEM_SHARED((N, D), jnp.float32),
        pltpu.SemaphoreType.DMA,
    ],
)
def _sc_scatter_add(h_hbm, src_hbm, dst_hbm, zero_hbm, out_hbm,
                    srci, dsti, rows, srct, dstt, rowt, accum, sem):
    c = lax.axis_index("c")
    s = lax.axis_index("s")
    base = (c * NS + s) * EPW

    @pl.when(s < NS - 1)
    def _():
        pltpu.sync_copy(zero_hbm.at[pl.ds(s * ROWS_MAIN, ROWS_MAIN)],
                        accum.at[pl.ds(s * ROWS_MAIN, ROWS_MAIN)])

    @pl.when(s == NS - 1)
    def _():
        pltpu.sync_copy(zero_hbm.at[pl.ds((NS - 1) * ROWS_MAIN, ROWS_LAST)],
                        accum.at[pl.ds((NS - 1) * ROWS_MAIN, ROWS_LAST)])

    plsc.subcore_barrier()

    def body(j, carry):
        eb = base + j * EDGE_B
        pltpu.sync_copy(src_hbm.at[pl.ds(eb, EDGE_B)], srci)
        pltpu.sync_copy(dst_hbm.at[pl.ds(eb, EDGE_B)], dsti)
        pltpu.async_copy(h_hbm.at[srci], rows, sem).wait()
        pltpu.sync_copy(rows, accum.at[dsti], add=True)
        return carry

    lax.fori_loop(0, NB_FULL, body, 0)

    et = base + NB_FULL * EDGE_B
    pltpu.sync_copy(src_hbm.at[pl.ds(et, TAIL)], srct)
    pltpu.sync_copy(dst_hbm.at[pl.ds(et, TAIL)], dstt)
    pltpu.async_copy(h_hbm.at[srct], rowt, sem).wait()
    pltpu.sync_copy(rowt, accum.at[dstt], add=True)

    plsc.subcore_barrier()

    @pl.when(s < NS - 1)
    def _():
        pltpu.sync_copy(accum.at[pl.ds(s * ROWS_MAIN, ROWS_MAIN)],
                        out_hbm.at[pl.ds(c * N + s * ROWS_MAIN, ROWS_MAIN)])

    @pl.when(s == NS - 1)
    def _():
        pltpu.sync_copy(
            accum.at[pl.ds((NS - 1) * ROWS_MAIN, ROWS_LAST)],
            out_hbm.at[pl.ds(c * N + (NS - 1) * ROWS_MAIN, ROWS_LAST)])


def kernel(x, edge_index, Wg1, bg1, We1, be1, bias1, Wg2, bg2, We2, be2,
           bias2, W3, bias3):
    src = edge_index[0]
    dst = edge_index[1]
    zeros = jnp.zeros((N, D), jnp.float32)

    h1, s1 = _moe_first(x, Wg1, bg1, We1, be1)
    p = _sc_scatter_add(h1, src, dst, zeros)
    h2, s2 = _moe_next(p[:N], p[N:], bias1, Wg2, bg2, We2, be2)
    p = _sc_scatter_add(h2, src, dst, zeros)
    h3 = _final_mm(p[:N], p[N:], bias2, W3)
    p = _sc_scatter_add(h3, src, dst, zeros)
    out = _combine(p[:N], p[N:], bias3)

    gate_std_means = jnp.stack([s1[0, 0], s2[0, 0]]) / N
    return out, gate_std_means
